# baseline (speedup 1.0000x reference)
import functools

import jax
import jax.numpy as jnp
from jax import lax
from jax.experimental import pallas as pl
from jax.experimental.pallas import tpu as pltpu
from jax.experimental.pallas import tpu_sc as plsc


def kernel(x, embed_table):
    b, _, h, w = x.shape
    hw = h * w
    d = embed_table.shape[1]

    info = plsc.get_sparse_core_info()
    nc, ns = info.num_cores, info.num_subcores
    nw = nc * ns
    nrb = 4
    rblk = hw // nrb
    nbg = nw // nrb
    bpw = b // nbg

    mesh = plsc.VectorSubcoreMesh(core_axis_name="c", subcore_axis_name="s")

    @functools.partial(
        pl.kernel,
        out_type=jax.ShapeDtypeStruct((b, hw, d), jnp.float32),
        mesh=mesh,
        scratch_types=[
            pltpu.VMEM((rblk, d), jnp.float32),
            pltpu.SemaphoreType.DMA,
        ],
    )
    def sc_broadcast(table_hbm, out_hbm, rows_v, sem):
        wid = lax.axis_index("s") * nc + lax.axis_index("c")
        rb = wid % nrb
        bg = wid // nrb
        pltpu.sync_copy(table_hbm.at[pl.ds(rb * rblk, rblk)], rows_v)
        for i in range(bpw):
            pltpu.async_copy(
                rows_v, out_hbm.at[bg + nbg * i, pl.ds(rb * rblk, rblk)], sem)
        for i in range(bpw):
            pltpu.make_async_copy(
                rows_v, out_hbm.at[bg + nbg * i, pl.ds(rb * rblk, rblk)],
                sem).wait()

    out = sc_broadcast(embed_table)
    return out.reshape(b, h, w, d).transpose(0, 3, 1, 2)


# speedup vs baseline: 3.5753x; 3.5753x over previous
import jax
import jax.numpy as jnp
from jax.experimental import pallas as pl


def kernel(x, embed_table):
    b, _, h, w = x.shape
    hw = h * w
    d = embed_table.shape[1]

    bb = 16
    grid = b // bb

    def body(e_ref, o_ref):
        o_ref[...] = jnp.broadcast_to(e_ref[...][None], (bb, hw, d))

    out = pl.pallas_call(
        body,
        grid=(grid,),
        in_specs=[pl.BlockSpec((hw, d), lambda i: (0, 0))],
        out_specs=pl.BlockSpec((bb, hw, d), lambda i: (i, 0, 0)),
        out_shape=jax.ShapeDtypeStruct((b, hw, d), embed_table.dtype),
    )(embed_table)
    return out.reshape(b, h, w, d).transpose(0, 3, 1, 2)
